# baseline (speedup 1.0000x reference)
import jax
import jax.numpy as jnp
from jax.experimental import pallas as pl
from jax.experimental.pallas import tpu as pltpu

_H = 128


def _dma_body(x_hbm, o_hbm, sem):
    for h in range(_H):
        pltpu.make_async_copy(
            x_hbm.at[:, h, :], o_hbm.at[:, _H - 1 - h, :], sem
        ).start()
    for h in range(_H):
        pltpu.make_async_copy(
            x_hbm.at[:, h, :], o_hbm.at[:, _H - 1 - h, :], sem
        ).wait()


def kernel(x):
    B, C, D, H, W = x.shape
    L = B * C * D
    xr = x.reshape(L, H, W)
    out = pl.pallas_call(
        _dma_body,
        in_specs=[pl.BlockSpec(memory_space=pltpu.MemorySpace.HBM)],
        out_specs=pl.BlockSpec(memory_space=pltpu.MemorySpace.HBM),
        out_shape=jax.ShapeDtypeStruct((L, H, W), x.dtype),
        scratch_shapes=[pltpu.SemaphoreType.DMA],
    )(xr)
    return out.reshape(B, C, D, H, W)


# speedup vs baseline: 43.2691x; 43.2691x over previous
import jax
import jax.numpy as jnp
from jax.experimental import pallas as pl

_HB = 8
_NG = 16


def _flip_body(x_ref, o_ref):
    for g in range(_NG):
        blk = x_ref[:, (_NG - 1 - g) * _HB : (_NG - g) * _HB, :]
        o_ref[:, g * _HB : (g + 1) * _HB, :] = jnp.concatenate(
            [blk[:, i : i + 1, :] for i in reversed(range(_HB))], axis=1
        )


def kernel(x):
    B, C, D, H, W = x.shape
    L = B * C * D
    xr = x.reshape(L, H, W)
    Lb = 64
    out = pl.pallas_call(
        _flip_body,
        grid=(L // Lb,),
        in_specs=[pl.BlockSpec((Lb, H, W), lambda l: (l, 0, 0))],
        out_specs=pl.BlockSpec((Lb, H, W), lambda l: (l, 0, 0)),
        out_shape=jax.ShapeDtypeStruct((L, H, W), x.dtype),
    )(xr)
    return out.reshape(B, C, D, H, W)
